# Initial kernel scaffold; baseline (speedup 1.0000x reference)
import jax
import jax.numpy as jnp
from jax.experimental import pallas as pl


def kernel(inputs, roi_label, lut, labels):
    raise NotImplementedError("write your pallas kernel here")


# speedup vs baseline: 1.4006x; 1.4006x over previous
import functools

import jax
import jax.numpy as jnp
from jax import lax
from jax.experimental import pallas as pl
from jax.experimental.pallas import tpu as pltpu
from jax.experimental.pallas import tpu_sc as plsc

_NUM_FEATURES = 256
_NUM_PIDS = 100000
_OIM_SCALAR = 30.0
_IGNORE_INDEX = 5554
_BATCH = 256

_BN = 4000
_K = _NUM_PIDS // _BN


def _make_sc_gather():
    info = plsc.get_sparse_core_info()
    nc, ns = info.num_cores, info.num_subcores
    nw = nc * ns
    b_per_w = _BATCH // nw
    mesh = plsc.VectorSubcoreMesh(core_axis_name="c", subcore_axis_name="s")

    @functools.partial(
        pl.kernel,
        mesh=mesh,
        out_type=[
            jax.ShapeDtypeStruct((_BATCH,), jnp.int32),
            jax.ShapeDtypeStruct((_BATCH, _NUM_FEATURES), jnp.float32),
        ],
        scratch_types=[
            pltpu.VMEM((b_per_w,), jnp.int32),
            pltpu.VMEM((b_per_w,), jnp.int32),
            pltpu.VMEM((b_per_w, _NUM_FEATURES), jnp.float32),
            pltpu.SemaphoreType.DMA,
        ],
    )
    def sc_gather(safe_hbm, labels_hbm, lut_hbm, label_out, rows_out,
                  idx_v, lbl_v, rows_v, sem):
        wid = lax.axis_index("s") * nc + lax.axis_index("c")
        base = wid * b_per_w
        pltpu.sync_copy(safe_hbm.at[pl.ds(base, b_per_w)], idx_v)
        pltpu.async_copy(labels_hbm.at[idx_v], lbl_v, sem).wait()
        pltpu.sync_copy(lbl_v, label_out.at[pl.ds(base, b_per_w)])
        pltpu.async_copy(lut_hbm.at[lbl_v], rows_v, sem).wait()
        pltpu.sync_copy(rows_v, rows_out.at[pl.ds(base, b_per_w)])

    return sc_gather


_sc_gather_cache = []


def _get_sc_gather():
    if not _sc_gather_cache:
        _sc_gather_cache.append(_make_sc_gather())
    return _sc_gather_cache[0]


def _tc_body(x_ref, lut_ref, glut_ref, label_ref, roi_ref, out_ref, m_s, s_s):
    k = pl.program_id(0)

    @pl.when(k == 0)
    def _init():
        m_s[...] = jnp.full((_BATCH, 1), -jnp.inf, jnp.float32)
        s_s[...] = jnp.zeros((_BATCH, 1), jnp.float32)

    proj = lax.dot_general(
        x_ref[...], lut_ref[...],
        (((1,), (1,)), ((), ())),
        preferred_element_type=jnp.float32,
    ) * _OIM_SCALAR
    bm = jnp.max(proj, axis=1, keepdims=True)
    m_new = jnp.maximum(m_s[...], bm)
    s_s[...] = s_s[...] * jnp.exp(m_s[...] - m_new) + jnp.sum(
        jnp.exp(proj - m_new), axis=1, keepdims=True)
    m_s[...] = m_new

    @pl.when(k == _K - 1)
    def _epilogue():
        logz = m_s[...] + jnp.log(s_s[...])
        picked = jnp.sum(x_ref[...] * glut_ref[...], axis=1,
                         keepdims=True) * _OIM_SCALAR
        nll = logz - picked
        label = label_ref[...]
        roi = roi_ref[...]
        valid = (roi - 1 >= 0) & (label != _IGNORE_INDEX)
        cnt = jnp.sum(valid.astype(jnp.float32))
        total = jnp.sum(jnp.where(valid, nll, 0.0))
        out_ref[...] = (total / jnp.maximum(cnt, 1.0)).reshape(1, 1)


_tc_loss = pl.pallas_call(
    _tc_body,
    grid=(_K,),
    in_specs=[
        pl.BlockSpec((_BATCH, _NUM_FEATURES), lambda k: (0, 0)),
        pl.BlockSpec((_BN, _NUM_FEATURES), lambda k: (k, 0)),
        pl.BlockSpec((_BATCH, _NUM_FEATURES), lambda k: (0, 0)),
        pl.BlockSpec((_BATCH, 1), lambda k: (0, 0)),
        pl.BlockSpec((_BATCH, 1), lambda k: (0, 0)),
    ],
    out_specs=pl.BlockSpec((1, 1), lambda k: (0, 0)),
    out_shape=jax.ShapeDtypeStruct((1, 1), jnp.float32),
    scratch_shapes=[
        pltpu.VMEM((_BATCH, 1), jnp.float32),
        pltpu.VMEM((_BATCH, 1), jnp.float32),
    ],
)


def kernel(inputs, roi_label, lut, labels):
    targets = roi_label - 1
    safe_targets = jnp.where(targets >= 0, targets, 0).astype(jnp.int32)
    label, glut = _get_sc_gather()(safe_targets, labels, lut)
    out = _tc_loss(inputs, lut, glut,
                   label.reshape(_BATCH, 1),
                   roi_label.reshape(_BATCH, 1))
    return out[0, 0]


# speedup vs baseline: 1.4259x; 1.0181x over previous
import functools

import jax
import jax.numpy as jnp
from jax import lax
from jax.experimental import pallas as pl
from jax.experimental.pallas import tpu as pltpu
from jax.experimental.pallas import tpu_sc as plsc

_NUM_FEATURES = 256
_NUM_PIDS = 100000
_OIM_SCALAR = 30.0
_IGNORE_INDEX = 5554
_BATCH = 256

_BN = 4000
_K = _NUM_PIDS // _BN


def _make_sc_gather():
    info = plsc.get_sparse_core_info()
    nc, ns = info.num_cores, info.num_subcores
    nw = nc * ns
    b_per_w = _BATCH // nw
    mesh = plsc.VectorSubcoreMesh(core_axis_name="c", subcore_axis_name="s")

    @functools.partial(
        pl.kernel,
        mesh=mesh,
        out_type=[
            jax.ShapeDtypeStruct((_BATCH,), jnp.int32),
            jax.ShapeDtypeStruct((_BATCH, _NUM_FEATURES), jnp.float32),
        ],
        scratch_types=[
            pltpu.VMEM((b_per_w,), jnp.int32),
            pltpu.VMEM((b_per_w,), jnp.int32),
            pltpu.VMEM((b_per_w, _NUM_FEATURES), jnp.float32),
            pltpu.SemaphoreType.DMA,
        ],
    )
    def sc_gather(safe_hbm, labels_hbm, lut_hbm, label_out, rows_out,
                  idx_v, lbl_v, rows_v, sem):
        wid = lax.axis_index("s") * nc + lax.axis_index("c")
        base = wid * b_per_w
        pltpu.sync_copy(safe_hbm.at[pl.ds(base, b_per_w)], idx_v)
        pltpu.async_copy(labels_hbm.at[idx_v], lbl_v, sem).wait()
        pltpu.sync_copy(lbl_v, label_out.at[pl.ds(base, b_per_w)])
        pltpu.async_copy(lut_hbm.at[lbl_v], rows_v, sem).wait()
        pltpu.sync_copy(rows_v, rows_out.at[pl.ds(base, b_per_w)])

    return sc_gather


_sc_gather_cache = []


def _get_sc_gather():
    if not _sc_gather_cache:
        _sc_gather_cache.append(_make_sc_gather())
    return _sc_gather_cache[0]


def _tc_body(xs_ref, x_ref, lut_ref, glut_ref, label_ref, roi_ref, out_ref,
             m_s, s_s):
    k = pl.program_id(0)

    @pl.when(k == 0)
    def _init():
        m_s[...] = jnp.full((_BATCH, 1), -jnp.inf, jnp.float32)
        s_s[...] = jnp.zeros((_BATCH, 1), jnp.float32)

    proj = lax.dot_general(
        xs_ref[...], lut_ref[...].astype(jnp.bfloat16),
        (((1,), (1,)), ((), ())),
        preferred_element_type=jnp.float32,
    )
    bm = jnp.max(proj, axis=1, keepdims=True)
    m_new = jnp.maximum(m_s[...], bm)
    s_s[...] = s_s[...] * jnp.exp(m_s[...] - m_new) + jnp.sum(
        jnp.exp(proj - m_new), axis=1, keepdims=True)
    m_s[...] = m_new

    @pl.when(k == _K - 1)
    def _epilogue():
        logz = m_s[...] + jnp.log(s_s[...])
        picked = jnp.sum(x_ref[...] * glut_ref[...], axis=1,
                         keepdims=True) * _OIM_SCALAR
        nll = logz - picked
        label = label_ref[...]
        roi = roi_ref[...]
        valid = (roi - 1 >= 0) & (label != _IGNORE_INDEX)
        cnt = jnp.sum(valid.astype(jnp.float32))
        total = jnp.sum(jnp.where(valid, nll, 0.0))
        out_ref[...] = (total / jnp.maximum(cnt, 1.0)).reshape(1, 1)


_tc_loss = pl.pallas_call(
    _tc_body,
    grid=(_K,),
    in_specs=[
        pl.BlockSpec((_BATCH, _NUM_FEATURES), lambda k: (0, 0)),
        pl.BlockSpec((_BATCH, _NUM_FEATURES), lambda k: (0, 0)),
        pl.BlockSpec((_BN, _NUM_FEATURES), lambda k: (k, 0)),
        pl.BlockSpec((_BATCH, _NUM_FEATURES), lambda k: (0, 0)),
        pl.BlockSpec((_BATCH, 1), lambda k: (0, 0)),
        pl.BlockSpec((_BATCH, 1), lambda k: (0, 0)),
    ],
    out_specs=pl.BlockSpec((1, 1), lambda k: (0, 0)),
    out_shape=jax.ShapeDtypeStruct((1, 1), jnp.float32),
    scratch_shapes=[
        pltpu.VMEM((_BATCH, 1), jnp.float32),
        pltpu.VMEM((_BATCH, 1), jnp.float32),
    ],
)


def kernel(inputs, roi_label, lut, labels):
    targets = roi_label - 1
    safe_targets = jnp.where(targets >= 0, targets, 0).astype(jnp.int32)
    label, glut = _get_sc_gather()(safe_targets, labels, lut)
    xs = (inputs * _OIM_SCALAR).astype(jnp.bfloat16)
    out = _tc_loss(xs, inputs, lut, glut,
                   label.reshape(_BATCH, 1),
                   roi_label.reshape(_BATCH, 1))
    return out[0, 0]


# speedup vs baseline: 1.4548x; 1.0202x over previous
import functools

import jax
import jax.numpy as jnp
from jax import lax
from jax.experimental import pallas as pl
from jax.experimental.pallas import tpu as pltpu
from jax.experimental.pallas import tpu_sc as plsc

_NUM_FEATURES = 256
_NUM_PIDS = 100000
_OIM_SCALAR = 30.0
_IGNORE_INDEX = 5554
_BATCH = 256

_BN = 4000
_K = _NUM_PIDS // _BN


def _make_sc_gather():
    info = plsc.get_sparse_core_info()
    nc, ns = info.num_cores, info.num_subcores
    nw = nc * ns
    b_per_w = _BATCH // nw
    mesh = plsc.VectorSubcoreMesh(core_axis_name="c", subcore_axis_name="s")

    @functools.partial(
        pl.kernel,
        mesh=mesh,
        out_type=[
            jax.ShapeDtypeStruct((_BATCH,), jnp.int32),
            jax.ShapeDtypeStruct((_BATCH, _NUM_FEATURES), jnp.float32),
        ],
        scratch_types=[
            pltpu.VMEM((b_per_w,), jnp.int32),
            pltpu.VMEM((b_per_w,), jnp.int32),
            pltpu.VMEM((b_per_w, _NUM_FEATURES), jnp.float32),
            pltpu.SemaphoreType.DMA,
        ],
    )
    def sc_gather(safe_hbm, labels_hbm, lut_hbm, label_out, rows_out,
                  idx_v, lbl_v, rows_v, sem):
        wid = lax.axis_index("s") * nc + lax.axis_index("c")
        base = wid * b_per_w
        pltpu.sync_copy(safe_hbm.at[pl.ds(base, b_per_w)], idx_v)
        pltpu.async_copy(labels_hbm.at[idx_v], lbl_v, sem).wait()
        pltpu.sync_copy(lbl_v, label_out.at[pl.ds(base, b_per_w)])
        pltpu.async_copy(lut_hbm.at[lbl_v], rows_v, sem).wait()
        pltpu.sync_copy(rows_v, rows_out.at[pl.ds(base, b_per_w)])

    return sc_gather


_sc_gather_cache = []


def _get_sc_gather():
    if not _sc_gather_cache:
        _sc_gather_cache.append(_make_sc_gather())
    return _sc_gather_cache[0]


_LOG2E = 1.4426950408889634
_LN2 = 0.6931471805599453


def _tc_body(x_ref, lut_ref, glut_ref, label_ref, roi_ref, out_ref, m_s, s_s):
    k = pl.program_id(0)

    @pl.when(k == 0)
    def _init():
        m_s[...] = jnp.full((_BATCH, 1), -jnp.inf, jnp.float32)
        s_s[...] = jnp.zeros((_BATCH, 1), jnp.float32)

    xbf = (x_ref[...] * (_OIM_SCALAR * _LOG2E)).astype(jnp.bfloat16)
    proj2 = lax.dot_general(
        xbf, lut_ref[...].astype(jnp.bfloat16),
        (((1,), (1,)), ((), ())),
        preferred_element_type=jnp.float32,
    )
    bm = jnp.max(proj2, axis=1, keepdims=True)
    m_new = jnp.maximum(m_s[...], bm)
    s_s[...] = s_s[...] * jnp.exp2(m_s[...] - m_new) + jnp.sum(
        jnp.exp2(proj2 - m_new), axis=1, keepdims=True)
    m_s[...] = m_new

    @pl.when(k == _K - 1)
    def _epilogue():
        logz = (m_s[...] + jnp.log2(s_s[...])) * _LN2
        picked = jnp.sum(x_ref[...] * glut_ref[...], axis=1,
                         keepdims=True) * _OIM_SCALAR
        nll = logz - picked
        label = label_ref[...]
        roi = roi_ref[...]
        valid = (roi - 1 >= 0) & (label != _IGNORE_INDEX)
        cnt = jnp.sum(valid.astype(jnp.float32))
        total = jnp.sum(jnp.where(valid, nll, 0.0))
        out_ref[...] = (total / jnp.maximum(cnt, 1.0)).reshape(1, 1)


_tc_loss = pl.pallas_call(
    _tc_body,
    grid=(_K,),
    in_specs=[
        pl.BlockSpec((_BATCH, _NUM_FEATURES), lambda k: (0, 0)),
        pl.BlockSpec((_BN, _NUM_FEATURES), lambda k: (k, 0)),
        pl.BlockSpec((_BATCH, _NUM_FEATURES), lambda k: (0, 0)),
        pl.BlockSpec((_BATCH, 1), lambda k: (0, 0)),
        pl.BlockSpec((_BATCH, 1), lambda k: (0, 0)),
    ],
    out_specs=pl.BlockSpec((1, 1), lambda k: (0, 0)),
    out_shape=jax.ShapeDtypeStruct((1, 1), jnp.float32),
    scratch_shapes=[
        pltpu.VMEM((_BATCH, 1), jnp.float32),
        pltpu.VMEM((_BATCH, 1), jnp.float32),
    ],
)


def kernel(inputs, roi_label, lut, labels):
    targets = roi_label - 1
    safe_targets = jnp.where(targets >= 0, targets, 0).astype(jnp.int32)
    label, glut = _get_sc_gather()(safe_targets, labels, lut)
    out = _tc_loss(inputs, lut, glut,
                   label.reshape(_BATCH, 1),
                   roi_label.reshape(_BATCH, 1))
    return out[0, 0]


# speedup vs baseline: 1.4931x; 1.0263x over previous
import functools

import jax
import jax.numpy as jnp
from jax import lax
from jax.experimental import pallas as pl
from jax.experimental.pallas import tpu as pltpu
from jax.experimental.pallas import tpu_sc as plsc

_NUM_FEATURES = 256
_NUM_PIDS = 100000
_OIM_SCALAR = 30.0
_IGNORE_INDEX = 5554
_BATCH = 256

_BN = 4000
_K = _NUM_PIDS // _BN


def _make_sc_gather():
    info = plsc.get_sparse_core_info()
    nc, ns = info.num_cores, info.num_subcores
    nw = nc * ns
    b_per_w = _BATCH // nw
    mesh = plsc.VectorSubcoreMesh(core_axis_name="c", subcore_axis_name="s")

    @functools.partial(
        pl.kernel,
        mesh=mesh,
        out_type=[
            jax.ShapeDtypeStruct((_BATCH,), jnp.int32),
            jax.ShapeDtypeStruct((_BATCH, _NUM_FEATURES), jnp.float32),
        ],
        scratch_types=[
            pltpu.VMEM((b_per_w,), jnp.int32),
            pltpu.VMEM((b_per_w,), jnp.int32),
            pltpu.VMEM((b_per_w, _NUM_FEATURES), jnp.float32),
            pltpu.SemaphoreType.DMA,
        ],
    )
    def sc_gather(safe_hbm, labels_hbm, lut_hbm, label_out, rows_out,
                  idx_v, lbl_v, rows_v, sem):
        wid = lax.axis_index("s") * nc + lax.axis_index("c")
        base = wid * b_per_w
        pltpu.sync_copy(safe_hbm.at[pl.ds(base, b_per_w)], idx_v)
        pltpu.async_copy(labels_hbm.at[idx_v], lbl_v, sem).wait()
        pltpu.sync_copy(lbl_v, label_out.at[pl.ds(base, b_per_w)])
        pltpu.async_copy(lut_hbm.at[lbl_v], rows_v, sem).wait()
        pltpu.sync_copy(rows_v, rows_out.at[pl.ds(base, b_per_w)])

    return sc_gather


_sc_gather_cache = []


def _get_sc_gather():
    if not _sc_gather_cache:
        _sc_gather_cache.append(_make_sc_gather())
    return _sc_gather_cache[0]


_LOG2E = 1.4426950408889634
_LN2 = 0.6931471805599453


def _tc_body(x_ref, lut_ref, logz_ref, m_s, s_s):
    k = pl.program_id(0)

    @pl.when(k == 0)
    def _init():
        m_s[...] = jnp.full((_BATCH, 1), -jnp.inf, jnp.float32)
        s_s[...] = jnp.zeros((_BATCH, 1), jnp.float32)

    xbf = (x_ref[...] * (_OIM_SCALAR * _LOG2E)).astype(jnp.bfloat16)
    proj2 = lax.dot_general(
        xbf, lut_ref[...].astype(jnp.bfloat16),
        (((1,), (1,)), ((), ())),
        preferred_element_type=jnp.float32,
    )
    bm = jnp.max(proj2, axis=1, keepdims=True)
    m_new = jnp.maximum(m_s[...], bm)
    s_s[...] = s_s[...] * jnp.exp2(m_s[...] - m_new) + jnp.sum(
        jnp.exp2(proj2 - m_new), axis=1, keepdims=True)
    m_s[...] = m_new

    @pl.when(k == _K - 1)
    def _finish():
        logz_ref[...] = (m_s[...] + jnp.log2(s_s[...])) * _LN2


_tc_logz = pl.pallas_call(
    _tc_body,
    grid=(_K,),
    in_specs=[
        pl.BlockSpec((_BATCH, _NUM_FEATURES), lambda k: (0, 0)),
        pl.BlockSpec((_BN, _NUM_FEATURES), lambda k: (k, 0)),
    ],
    out_specs=pl.BlockSpec((_BATCH, 1), lambda k: (0, 0)),
    out_shape=jax.ShapeDtypeStruct((_BATCH, 1), jnp.float32),
    scratch_shapes=[
        pltpu.VMEM((_BATCH, 1), jnp.float32),
        pltpu.VMEM((_BATCH, 1), jnp.float32),
    ],
)


def _epi_body(x_ref, glut_ref, label_ref, roi_ref, logz_ref, out_ref):
    picked = jnp.sum(x_ref[...] * glut_ref[...], axis=1,
                     keepdims=True) * _OIM_SCALAR
    nll = logz_ref[...] - picked
    valid = (roi_ref[...] - 1 >= 0) & (label_ref[...] != _IGNORE_INDEX)
    cnt = jnp.sum(valid.astype(jnp.float32))
    total = jnp.sum(jnp.where(valid, nll, 0.0))
    out_ref[...] = (total / jnp.maximum(cnt, 1.0)).reshape(1, 1)


_tc_epi = pl.pallas_call(
    _epi_body,
    out_shape=jax.ShapeDtypeStruct((1, 1), jnp.float32),
)


def kernel(inputs, roi_label, lut, labels):
    targets = roi_label - 1
    safe_targets = jnp.where(targets >= 0, targets, 0).astype(jnp.int32)
    label, glut = _get_sc_gather()(safe_targets, labels, lut)
    logz = _tc_logz(inputs, lut)
    out = _tc_epi(inputs, glut,
                  label.reshape(_BATCH, 1),
                  roi_label.reshape(_BATCH, 1),
                  logz)
    return out[0, 0]


# speedup vs baseline: 1.7256x; 1.1558x over previous
import functools

import jax
import jax.numpy as jnp
from jax import lax
from jax.experimental import pallas as pl
from jax.experimental.pallas import tpu as pltpu
from jax.experimental.pallas import tpu_sc as plsc

_NUM_FEATURES = 256
_NUM_PIDS = 100000
_OIM_SCALAR = 30.0
_IGNORE_INDEX = 5554
_BATCH = 256

_BN = 10000
_K = _NUM_PIDS // _BN


def _make_sc_gather():
    info = plsc.get_sparse_core_info()
    nc, ns = info.num_cores, info.num_subcores
    nw = nc * ns
    b_per_w = _BATCH // nw
    mesh = plsc.VectorSubcoreMesh(core_axis_name="c", subcore_axis_name="s")

    @functools.partial(
        pl.kernel,
        mesh=mesh,
        out_type=[
            jax.ShapeDtypeStruct((_BATCH,), jnp.int32),
            jax.ShapeDtypeStruct((_BATCH, _NUM_FEATURES), jnp.float32),
        ],
        scratch_types=[
            pltpu.VMEM((b_per_w,), jnp.int32),
            pltpu.VMEM((b_per_w,), jnp.int32),
            pltpu.VMEM((b_per_w, _NUM_FEATURES), jnp.float32),
            pltpu.SemaphoreType.DMA,
        ],
    )
    def sc_gather(safe_hbm, labels_hbm, lut_hbm, label_out, rows_out,
                  idx_v, lbl_v, rows_v, sem):
        wid = lax.axis_index("s") * nc + lax.axis_index("c")
        base = wid * b_per_w
        pltpu.sync_copy(safe_hbm.at[pl.ds(base, b_per_w)], idx_v)
        pltpu.async_copy(labels_hbm.at[idx_v], lbl_v, sem).wait()
        pltpu.sync_copy(lbl_v, label_out.at[pl.ds(base, b_per_w)])
        pltpu.async_copy(lut_hbm.at[lbl_v], rows_v, sem).wait()
        pltpu.sync_copy(rows_v, rows_out.at[pl.ds(base, b_per_w)])

    return sc_gather


_sc_gather_cache = []


def _get_sc_gather():
    if not _sc_gather_cache:
        _sc_gather_cache.append(_make_sc_gather())
    return _sc_gather_cache[0]


_LOG2E = 1.4426950408889634
_LN2 = 0.6931471805599453


def _tc_body(x_ref, lut_ref, logz_ref, m_s, s_s):
    k = pl.program_id(0)

    @pl.when(k == 0)
    def _init():
        m_s[...] = jnp.full((_BATCH, 1), -jnp.inf, jnp.float32)
        s_s[...] = jnp.zeros((_BATCH, 1), jnp.float32)

    xbf = (x_ref[...] * (_OIM_SCALAR * _LOG2E)).astype(jnp.bfloat16)
    proj2 = lax.dot_general(
        xbf, lut_ref[...].astype(jnp.bfloat16),
        (((1,), (1,)), ((), ())),
        preferred_element_type=jnp.float32,
    )
    bm = jnp.max(proj2, axis=1, keepdims=True)
    m_new = jnp.maximum(m_s[...], bm)
    s_s[...] = s_s[...] * jnp.exp2(m_s[...] - m_new) + jnp.sum(
        jnp.exp2(proj2 - m_new), axis=1, keepdims=True)
    m_s[...] = m_new

    @pl.when(k == _K - 1)
    def _finish():
        logz_ref[...] = (m_s[...] + jnp.log2(s_s[...])) * _LN2


_tc_logz = pl.pallas_call(
    _tc_body,
    grid=(_K,),
    in_specs=[
        pl.BlockSpec((_BATCH, _NUM_FEATURES), lambda k: (0, 0)),
        pl.BlockSpec((_BN, _NUM_FEATURES), lambda k: (k, 0)),
    ],
    out_specs=pl.BlockSpec((_BATCH, 1), lambda k: (0, 0)),
    out_shape=jax.ShapeDtypeStruct((_BATCH, 1), jnp.float32),
    scratch_shapes=[
        pltpu.VMEM((_BATCH, 1), jnp.float32),
        pltpu.VMEM((_BATCH, 1), jnp.float32),
    ],
)


def _epi_body(x_ref, glut_ref, label_ref, roi_ref, logz_ref, out_ref):
    picked = jnp.sum(x_ref[...] * glut_ref[...], axis=1,
                     keepdims=True) * _OIM_SCALAR
    nll = logz_ref[...] - picked
    valid = (roi_ref[...] - 1 >= 0) & (label_ref[...] != _IGNORE_INDEX)
    cnt = jnp.sum(valid.astype(jnp.float32))
    total = jnp.sum(jnp.where(valid, nll, 0.0))
    out_ref[...] = (total / jnp.maximum(cnt, 1.0)).reshape(1, 1)


_tc_epi = pl.pallas_call(
    _epi_body,
    out_shape=jax.ShapeDtypeStruct((1, 1), jnp.float32),
)


def kernel(inputs, roi_label, lut, labels):
    targets = roi_label - 1
    safe_targets = jnp.where(targets >= 0, targets, 0).astype(jnp.int32)
    label, glut = _get_sc_gather()(safe_targets, labels, lut)
    logz = _tc_logz(inputs, lut)
    out = _tc_epi(inputs, glut,
                  label.reshape(_BATCH, 1),
                  roi_label.reshape(_BATCH, 1),
                  logz)
    return out[0, 0]
